# baseline (speedup 1.0000x reference)
import functools

import jax
import jax.numpy as jnp
from jax import lax
from jax.experimental import pallas as pl
from jax.experimental.pallas import tpu as pltpu
from jax.experimental.pallas import tpu_sc as plsc

_D = 64
_CHUNK = 256
_NW = 32
_BPW = 512


def _embed_lookup(ids_t, table2, S, B):
    n_sub = _BPW // _CHUNK
    mesh = plsc.VectorSubcoreMesh(core_axis_name="c", subcore_axis_name="s")

    @functools.partial(
        pl.kernel,
        out_type=jax.ShapeDtypeStruct((S, _D, B), jnp.float32),
        mesh=mesh,
        scratch_types=[
            pltpu.VMEM((S, _BPW), jnp.int32),
            pltpu.VMEM((2, 2, 128), jnp.int32),
            pltpu.VMEM((2, _CHUNK, 128), jnp.float32),
            pltpu.VMEM((2, _D, _CHUNK), jnp.float32),
            pltpu.SemaphoreType.DMA,
            pltpu.SemaphoreType.DMA((2,)),
            pltpu.SemaphoreType.DMA((2,)),
        ],
        compiler_params=pltpu.CompilerParams(
            use_tc_tiling_on_sc=True, needs_layout_passes=False),
    )
    def body(ids_hbm, table_hbm, out_hbm, ids_v, pidx, gbuf, obuf,
             sem_i, sem_g, sem_w):
        wid = lax.axis_index("s") * 2 + lax.axis_index("c")
        b_lo = wid * _BPW
        iota16 = lax.iota(jnp.int32, 16)

        pltpu.async_copy(ids_hbm.at[:, pl.ds(b_lo, _BPW)], ids_v, sem_i).wait()

        def build_pidx(s, b):
            def g_step(g, c):
                v16 = ids_v[s, pl.ds(b * _CHUNK + g * 16, 16)]
                pidx[b, g // 8, pl.ds((g % 8) * 16, 16)] = v16 >> 1
                return c
            lax.fori_loop(0, 16, g_step, 0)

        def start_gather(b):
            for h in range(2):
                pltpu.async_copy(table_hbm.at[pidx.at[b, h]],
                                 gbuf.at[b, pl.ds(h * 128, 128)],
                                 sem_g.at[b])

        def wait_gather(b):
            for h in range(2):
                pltpu.make_async_copy(table_hbm.at[pidx.at[b, h]],
                                      gbuf.at[b, pl.ds(h * 128, 128)],
                                      sem_g.at[b]).wait()

        def start_write(s, b):
            pltpu.async_copy(
                obuf.at[b],
                out_hbm.at[s, :, pl.ds(b_lo + b * _CHUNK, _CHUNK)],
                sem_w.at[b])

        def wait_write(b):
            pltpu.make_async_copy(
                obuf.at[b], out_hbm.at[0, :, pl.ds(b_lo, _CHUNK)],
                sem_w.at[b]).wait()

        def transpose(s, b):
            def g_step(g, c):
                v16 = ids_v[s, pl.ds(b * _CHUNK + g * 16, 16)]
                row16 = g * 16 + iota16
                col0 = (v16 & 1) * 64
                for d in range(_D):
                    vals = plsc.load_gather(gbuf.at[b], [row16, col0 + d])
                    obuf[b, d, pl.ds(g * 16, 16)] = vals
                return c
            lax.fori_loop(0, 16, g_step, 0)

        build_pidx(0, 0)
        start_gather(0)

        def outer(s, carry):
            for b in range(n_sub):
                nb = 1 - b
                ns = lax.min(s + b, S - 1)
                build_pidx(ns, nb)
                start_gather(nb)
                wait_gather(b)
                @pl.when(s > 0)
                def _():
                    wait_write(b)
                transpose(s, b)
                start_write(s, b)
            return carry

        lax.fori_loop(0, S, outer, 0)

        wait_gather(0)
        wait_write(0)
        wait_write(1)

    return body(ids_t, table2)


def kernel(token_ids, weight):
    B0, B1 = token_ids.shape
    ids_t = token_ids.T.astype(jnp.int32)
    table2 = weight.reshape(-1, 128)
    out = _embed_lookup(ids_t, table2, B1, B0)
    return jnp.transpose(out, (2, 0, 1))


# speedup vs baseline: 1.4706x; 1.4706x over previous
import functools

import jax
import jax.numpy as jnp
from jax import lax
from jax.experimental import pallas as pl
from jax.experimental.pallas import tpu as pltpu
from jax.experimental.pallas import tpu_sc as plsc

_D = 64
_CHUNK = 128
_NBUF = 4
_NW = 32


def _embed_lookup(flat_ids, weight):
    B = flat_ids.shape[0]
    assert B % (_NW * _CHUNK * _NBUF) == 0
    n_chunks = B // (_NW * _CHUNK)
    n_outer = n_chunks // _NBUF
    idx2d = flat_ids.reshape(B // _CHUNK, _CHUNK)

    mesh = plsc.VectorSubcoreMesh(core_axis_name="c", subcore_axis_name="s")

    @functools.partial(
        pl.kernel,
        out_type=jax.ShapeDtypeStruct((B, _D), jnp.float32),
        mesh=mesh,
        scratch_types=[
            pltpu.VMEM((n_chunks, _CHUNK), jnp.int32),
            pltpu.VMEM((_NBUF, _CHUNK, _D), jnp.float32),
            pltpu.SemaphoreType.DMA((_NBUF,)),
            pltpu.SemaphoreType.DMA((_NBUF,)),
        ],
        compiler_params=pltpu.CompilerParams(use_tc_tiling_on_sc=False),
    )
    def body(idx_hbm, table_hbm, out_hbm, idx_v, rows_v, sem_g, sem_w):
        wid = lax.axis_index("s") * 2 + lax.axis_index("c")
        row_base = wid * n_chunks
        out_base = wid * n_chunks * _CHUNK
        pltpu.sync_copy(idx_hbm.at[pl.ds(row_base, n_chunks)], idx_v)

        def start_gather(j, b):
            pltpu.async_copy(table_hbm.at[idx_v.at[j]], rows_v.at[b],
                             sem_g.at[b])

        def wait_gather(j, b):
            pltpu.make_async_copy(table_hbm.at[idx_v.at[j]], rows_v.at[b],
                                  sem_g.at[b]).wait()

        def start_write(j, b):
            pltpu.async_copy(
                rows_v.at[b],
                out_hbm.at[pl.ds(out_base + j * _CHUNK, _CHUNK)],
                sem_w.at[b])

        def wait_write(b):
            pltpu.make_async_copy(
                rows_v.at[b], out_hbm.at[pl.ds(out_base, _CHUNK)],
                sem_w.at[b]).wait()

        for b in range(_NBUF):
            start_gather(b, b)

        def outer(g, carry):
            for b in range(_NBUF):
                j = g * _NBUF + b
                wait_gather(j, b)
                start_write(j, b)
            for b in range(_NBUF):
                jn = (g + 1) * _NBUF + b
                wait_write(b)
                start_gather(jn, b)
            return carry

        lax.fori_loop(0, n_outer - 1, outer, 0)

        gl = n_outer - 1
        for b in range(_NBUF):
            j = gl * _NBUF + b
            wait_gather(j, b)
            start_write(j, b)
        for b in range(_NBUF):
            wait_write(b)

    return body(idx2d, weight)


def kernel(token_ids, weight):
    B0, B1 = token_ids.shape
    flat = token_ids.reshape(-1).astype(jnp.int32)
    out = _embed_lookup(flat, weight)
    return out.reshape(B0, B1, _D)


# speedup vs baseline: 1.6145x; 1.0978x over previous
import functools

import jax
import jax.numpy as jnp
from jax import lax
from jax.experimental import pallas as pl
from jax.experimental.pallas import tpu as pltpu
from jax.experimental.pallas import tpu_sc as plsc

_D = 64
_CHUNK = 128
_NBUF = 4
_NW = 32
_V = 1000000
_VB = 8192
_TB = 1024
_S = 50
_B = 16384


def _pack_table(wt):
    def body(x_ref, y_ref):
        x = x_ref[...]
        xt = jnp.transpose(x)
        y_ref[...] = jnp.concatenate(
            [xt, jnp.zeros((_VB, 128 - _D), jnp.float32)], axis=1)

    return pl.pallas_call(
        body,
        grid=(pl.cdiv(_V, _VB),),
        in_specs=[pl.BlockSpec((_D, _VB), lambda j: (0, j))],
        out_specs=pl.BlockSpec((_VB, 128), lambda j: (j, 0)),
        out_shape=jax.ShapeDtypeStruct((_V, 128), jnp.float32),
    )(wt)


def _unpack_out(g3):
    def body(x_ref, y_ref):
        x = x_ref[0]
        y_ref[0] = jnp.transpose(x[:, :_D])

    return pl.pallas_call(
        body,
        grid=(_S, _B // _TB),
        in_specs=[pl.BlockSpec((1, _TB, 128), lambda s, j: (s, j, 0))],
        out_specs=pl.BlockSpec((1, _D, _TB), lambda s, j: (s, 0, j)),
        out_shape=jax.ShapeDtypeStruct((_S, _D, _B), jnp.float32),
    )(g3)


def _embed_lookup(flat_ids, table):
    B = flat_ids.shape[0]
    assert B % (_NW * _CHUNK * _NBUF) == 0
    n_chunks = B // (_NW * _CHUNK)
    n_outer = n_chunks // _NBUF
    idx2d = flat_ids.reshape(B // _CHUNK, _CHUNK)

    mesh = plsc.VectorSubcoreMesh(core_axis_name="c", subcore_axis_name="s")

    @functools.partial(
        pl.kernel,
        out_type=jax.ShapeDtypeStruct((B, 128), jnp.float32),
        mesh=mesh,
        scratch_types=[
            pltpu.VMEM((n_chunks, _CHUNK), jnp.int32),
            pltpu.VMEM((_NBUF, _CHUNK, 128), jnp.float32),
            pltpu.SemaphoreType.DMA((_NBUF,)),
            pltpu.SemaphoreType.DMA((_NBUF,)),
        ],
        compiler_params=pltpu.CompilerParams(use_tc_tiling_on_sc=False),
    )
    def body(idx_hbm, table_hbm, out_hbm, idx_v, rows_v, sem_g, sem_w):
        wid = lax.axis_index("s") * 2 + lax.axis_index("c")
        row_base = wid * n_chunks
        out_base = wid * n_chunks * _CHUNK
        pltpu.sync_copy(idx_hbm.at[pl.ds(row_base, n_chunks)], idx_v)

        def start_gather(j, b):
            pltpu.async_copy(table_hbm.at[idx_v.at[j]], rows_v.at[b],
                             sem_g.at[b])

        def wait_gather(j, b):
            pltpu.make_async_copy(table_hbm.at[idx_v.at[j]], rows_v.at[b],
                                  sem_g.at[b]).wait()

        def start_write(j, b):
            pltpu.async_copy(
                rows_v.at[b],
                out_hbm.at[pl.ds(out_base + j * _CHUNK, _CHUNK)],
                sem_w.at[b])

        def wait_write(b):
            pltpu.make_async_copy(
                rows_v.at[b], out_hbm.at[pl.ds(out_base, _CHUNK)],
                sem_w.at[b]).wait()

        for b in range(_NBUF):
            start_gather(b, b)

        def outer(g, carry):
            for b in range(_NBUF):
                j = g * _NBUF + b
                wait_gather(j, b)
                start_write(j, b)
            for b in range(_NBUF):
                jn = (g + 1) * _NBUF + b
                wait_write(b)
                start_gather(jn, b)
            return carry

        lax.fori_loop(0, n_outer - 1, outer, 0)

        gl = n_outer - 1
        for b in range(_NBUF):
            j = gl * _NBUF + b
            wait_gather(j, b)
            start_write(j, b)
        for b in range(_NBUF):
            wait_write(b)

    return body(idx2d, table)


def kernel(token_ids, weight):
    ids_flat = token_ids.T.astype(jnp.int32).reshape(-1)
    table = _pack_table(weight.T)
    g = _embed_lookup(ids_flat, table)
    out_t = _unpack_out(g.reshape(_S, _B, 128))
    return jnp.transpose(out_t, (2, 0, 1))


# speedup vs baseline: 1.6223x; 1.0049x over previous
import functools

import jax
import jax.numpy as jnp
from jax import lax
from jax.experimental import pallas as pl
from jax.experimental.pallas import tpu as pltpu
from jax.experimental.pallas import tpu_sc as plsc

_D = 64
_CHUNK = 128
_NBUF = 4
_NW = 32
_V = 1000000
_VB = 8192
_TB = 1024
_S = 50
_B = 16384


def _pack_table(wt):
    def body(x_ref, y_ref):
        x = x_ref[...]
        xt = jnp.transpose(x)
        y_ref[...] = jnp.concatenate(
            [xt, jnp.zeros((_VB, 128 - _D), jnp.float32)], axis=1)

    return pl.pallas_call(
        body,
        grid=(pl.cdiv(_V, _VB),),
        in_specs=[pl.BlockSpec((_D, _VB), lambda j: (0, j))],
        out_specs=pl.BlockSpec((_VB, 128), lambda j: (j, 0)),
        out_shape=jax.ShapeDtypeStruct((_V, 128), jnp.float32),
    )(wt)


def _unpack_out(g3):
    def body(x_ref, y_ref):
        x = x_ref[0]
        y_ref[0] = jnp.concatenate(
            [jnp.transpose(x[:, :_D]), jnp.transpose(x[:, _D:])], axis=1)

    return pl.pallas_call(
        body,
        grid=(_S, _B // _TB),
        in_specs=[pl.BlockSpec((1, _TB // 2, 128), lambda s, j: (s, j, 0))],
        out_specs=pl.BlockSpec((1, _D, _TB), lambda s, j: (s, 0, j)),
        out_shape=jax.ShapeDtypeStruct((_S, _D, _B), jnp.float32),
    )(g3)


def _embed_lookup(flat_ids, table):
    B = flat_ids.shape[0]
    assert B % (_NW * _CHUNK * _NBUF) == 0
    n_chunks = B // (_NW * _CHUNK)
    n_outer = n_chunks // _NBUF
    idx2d = flat_ids.reshape(B // _CHUNK, _CHUNK)

    mesh = plsc.VectorSubcoreMesh(core_axis_name="c", subcore_axis_name="s")

    @functools.partial(
        pl.kernel,
        out_type=jax.ShapeDtypeStruct((B // 2, 128), jnp.float32),
        mesh=mesh,
        scratch_types=[
            pltpu.VMEM((n_chunks, _CHUNK), jnp.int32),
            pltpu.VMEM((_NBUF, _CHUNK, 128), jnp.float32),
            pltpu.SemaphoreType.DMA((_NBUF,)),
            pltpu.SemaphoreType.DMA((_NBUF,)),
        ],
        compiler_params=pltpu.CompilerParams(use_tc_tiling_on_sc=False),
    )
    def body(idx_hbm, table_hbm, out_hbm, idx_v, rows_v, sem_g, sem_w):
        wid = lax.axis_index("s") * 2 + lax.axis_index("c")
        row_base = wid * n_chunks
        pltpu.sync_copy(idx_hbm.at[pl.ds(row_base, n_chunks)], idx_v)

        def dst_slot(j):
            c = row_base + j
            s = c >> 7
            b0 = (c << 7) & (_B - 1)
            jblk = b0 >> 10
            k0 = b0 & 511
            h = (b0 >> 9) & 1
            return s * (_B // 2) + jblk * 512 + k0, h

        def start_gather(j, b):
            pltpu.async_copy(table_hbm.at[idx_v.at[j]], rows_v.at[b],
                             sem_g.at[b])

        def wait_gather(j, b):
            pltpu.make_async_copy(table_hbm.at[idx_v.at[j]], rows_v.at[b],
                                  sem_g.at[b]).wait()

        def start_write(j, b):
            r, h = dst_slot(j)
            pltpu.async_copy(
                rows_v.at[b, :, pl.ds(0, _D)],
                out_hbm.at[pl.ds(r, _CHUNK), pl.ds(h * _D, _D)],
                sem_w.at[b])

        def wait_write(b):
            pltpu.make_async_copy(
                rows_v.at[b, :, pl.ds(0, _D)],
                out_hbm.at[pl.ds(0, _CHUNK), pl.ds(0, _D)],
                sem_w.at[b]).wait()

        for b in range(_NBUF):
            start_gather(b, b)

        def outer(g, carry):
            for b in range(_NBUF):
                j = g * _NBUF + b
                wait_gather(j, b)
                start_write(j, b)
            for b in range(_NBUF):
                jn = (g + 1) * _NBUF + b
                wait_write(b)
                start_gather(jn, b)
            return carry

        lax.fori_loop(0, n_outer - 1, outer, 0)

        gl = n_outer - 1
        for b in range(_NBUF):
            j = gl * _NBUF + b
            wait_gather(j, b)
            start_write(j, b)
        for b in range(_NBUF):
            wait_write(b)

    return body(idx2d, table)


def kernel(token_ids, weight):
    ids_flat = token_ids.T.astype(jnp.int32).reshape(-1)
    table = _pack_table(weight.T)
    g = _embed_lookup(ids_flat, table)
    out_t = _unpack_out(g.reshape(_S, _B // 2, 128))
    return jnp.transpose(out_t, (2, 0, 1))


# speedup vs baseline: 2.3989x; 1.4787x over previous
import functools

import jax
import jax.numpy as jnp
from jax import lax
from jax.experimental import pallas as pl
from jax.experimental.pallas import tpu as pltpu
from jax.experimental.pallas import tpu_sc as plsc

_D = 64
_CHUNK = 128
_NBUF = 4
_NW = 32
_V = 1000000
_VB = 16384
_TB = 8192
_S = 50
_B = 16384


def _pack_table(wt):
    def body(x_ref, y_ref):
        x = x_ref[...]
        xt = jnp.transpose(x)
        y_ref[...] = jnp.concatenate(
            [xt, jnp.zeros((_VB, 128 - _D), jnp.float32)], axis=1)

    return pl.pallas_call(
        body,
        grid=(pl.cdiv(_V, _VB),),
        in_specs=[pl.BlockSpec((_D, _VB), lambda j: (0, j))],
        out_specs=pl.BlockSpec((_VB, 128), lambda j: (j, 0)),
        out_shape=jax.ShapeDtypeStruct((_V, 128), jnp.float32),
    )(wt)


def _unpack_out(g3):
    def body(x_ref, y_ref):
        x = x_ref[0]
        y_ref[0] = jnp.concatenate(
            [jnp.transpose(x[:, :_D]), jnp.transpose(x[:, _D:])], axis=1)

    return pl.pallas_call(
        body,
        grid=(_S, _B // _TB),
        in_specs=[pl.BlockSpec((1, _TB // 2, 128), lambda s, j: (s, j, 0))],
        out_specs=pl.BlockSpec((1, _D, _TB), lambda s, j: (s, 0, j)),
        out_shape=jax.ShapeDtypeStruct((_S, _D, _B), jnp.float32),
    )(g3)


def _embed_lookup(flat_ids, table):
    B = flat_ids.shape[0]
    assert B % (_NW * _CHUNK * _NBUF) == 0
    n_chunks = B // (_NW * _CHUNK)
    n_outer = n_chunks // _NBUF
    idx2d = flat_ids.reshape(B // _CHUNK, _CHUNK)

    mesh = plsc.VectorSubcoreMesh(core_axis_name="c", subcore_axis_name="s")

    @functools.partial(
        pl.kernel,
        out_type=jax.ShapeDtypeStruct((B // 2, 128), jnp.float32),
        mesh=mesh,
        scratch_types=[
            pltpu.VMEM((n_chunks, _CHUNK), jnp.int32),
            pltpu.VMEM((_NBUF, _CHUNK, 128), jnp.float32),
            pltpu.SemaphoreType.DMA((_NBUF,)),
            pltpu.SemaphoreType.DMA((_NBUF,)),
        ],
        compiler_params=pltpu.CompilerParams(use_tc_tiling_on_sc=False),
    )
    def body(idx_hbm, table_hbm, out_hbm, idx_v, rows_v, sem_g, sem_w):
        wid = lax.axis_index("s") * 2 + lax.axis_index("c")
        row_base = wid * n_chunks
        pltpu.sync_copy(idx_hbm.at[pl.ds(row_base, n_chunks)], idx_v)

        def dst_slot(j):
            half = _TB // 2
            c = row_base + j
            s = c >> 7
            b0 = (c << 7) & (_B - 1)
            jblk = b0 // _TB
            k0 = b0 % half
            h = (b0 // half) & 1
            return s * (_B // 2) + jblk * half + k0, h

        def start_gather(j, b):
            pltpu.async_copy(table_hbm.at[idx_v.at[j]], rows_v.at[b],
                             sem_g.at[b])

        def wait_gather(j, b):
            pltpu.make_async_copy(table_hbm.at[idx_v.at[j]], rows_v.at[b],
                                  sem_g.at[b]).wait()

        def start_write(j, b):
            r, h = dst_slot(j)
            pltpu.async_copy(
                rows_v.at[b, :, pl.ds(0, _D)],
                out_hbm.at[pl.ds(r, _CHUNK), pl.ds(h * _D, _D)],
                sem_w.at[b])

        def wait_write(b):
            pltpu.make_async_copy(
                rows_v.at[b, :, pl.ds(0, _D)],
                out_hbm.at[pl.ds(0, _CHUNK), pl.ds(0, _D)],
                sem_w.at[b]).wait()

        for b in range(_NBUF):
            start_gather(b, b)

        def outer(g, carry):
            for b in range(_NBUF):
                j = g * _NBUF + b
                wait_gather(j, b)
                start_write(j, b)
            for b in range(_NBUF):
                jn = (g + 1) * _NBUF + b
                wait_write(b)
                start_gather(jn, b)
            return carry

        lax.fori_loop(0, n_outer - 1, outer, 0)

        gl = n_outer - 1
        for b in range(_NBUF):
            j = gl * _NBUF + b
            wait_gather(j, b)
            start_write(j, b)
        for b in range(_NBUF):
            wait_write(b)

    return body(idx2d, table)


def kernel(token_ids, weight):
    ids_flat = token_ids.T.astype(jnp.int32).reshape(-1)
    table = _pack_table(weight.T)
    g = _embed_lookup(ids_flat, table)
    out_t = _unpack_out(g.reshape(_S, _B // 2, 128))
    return jnp.transpose(out_t, (2, 0, 1))


# speedup vs baseline: 2.4895x; 1.0378x over previous
import functools

import jax
import jax.numpy as jnp
from jax import lax
from jax.experimental import pallas as pl
from jax.experimental.pallas import tpu as pltpu
from jax.experimental.pallas import tpu_sc as plsc

_D = 64
_CHUNK = 128
_NBUF = 4
_NW = 32
_V = 1000000
_VB = 16384
_TB = 16384
_S = 50
_B = 16384


def _pack_table(wt):
    def body(x_ref, y_ref):
        x = x_ref[...]
        xt = jnp.transpose(x)
        y_ref[...] = jnp.concatenate(
            [xt, jnp.zeros((_VB, 128 - _D), jnp.float32)], axis=1)

    return pl.pallas_call(
        body,
        grid=(pl.cdiv(_V, _VB),),
        in_specs=[pl.BlockSpec((_D, _VB), lambda j: (0, j))],
        out_specs=pl.BlockSpec((_VB, 128), lambda j: (j, 0)),
        out_shape=jax.ShapeDtypeStruct((_V, 128), jnp.float32),
    )(wt)


def _unpack_out(g3):
    def body(x_ref, y_ref):
        x = x_ref[0]
        y_ref[0] = jnp.concatenate(
            [jnp.transpose(x[:, :_D]), jnp.transpose(x[:, _D:])], axis=1)

    return pl.pallas_call(
        body,
        grid=(_S, _B // _TB),
        in_specs=[pl.BlockSpec((1, _TB // 2, 128), lambda s, j: (s, j, 0))],
        out_specs=pl.BlockSpec((1, _D, _TB), lambda s, j: (s, 0, j)),
        out_shape=jax.ShapeDtypeStruct((_S, _D, _B), jnp.float32),
    )(g3)


def _embed_lookup(flat_ids, table):
    B = flat_ids.shape[0]
    assert B % (_NW * _CHUNK * _NBUF) == 0
    n_chunks = B // (_NW * _CHUNK)
    n_outer = n_chunks // _NBUF
    idx2d = flat_ids.reshape(B // _CHUNK, _CHUNK)

    mesh = plsc.VectorSubcoreMesh(core_axis_name="c", subcore_axis_name="s")

    @functools.partial(
        pl.kernel,
        out_type=jax.ShapeDtypeStruct((B // 2, 128), jnp.float32),
        mesh=mesh,
        scratch_types=[
            pltpu.VMEM((n_chunks, _CHUNK), jnp.int32),
            pltpu.VMEM((_NBUF, _CHUNK, 128), jnp.float32),
            pltpu.SemaphoreType.DMA((_NBUF,)),
            pltpu.SemaphoreType.DMA((_NBUF,)),
        ],
        compiler_params=pltpu.CompilerParams(use_tc_tiling_on_sc=False),
    )
    def body(idx_hbm, table_hbm, out_hbm, idx_v, rows_v, sem_g, sem_w):
        wid = lax.axis_index("s") * 2 + lax.axis_index("c")
        row_base = wid * n_chunks
        pltpu.sync_copy(idx_hbm.at[pl.ds(row_base, n_chunks)], idx_v)

        def dst_slot(j):
            half = _TB // 2
            c = row_base + j
            s = c >> 7
            b0 = (c << 7) & (_B - 1)
            jblk = b0 // _TB
            k0 = b0 % half
            h = (b0 // half) & 1
            return s * (_B // 2) + jblk * half + k0, h

        def start_gather(j, b):
            pltpu.async_copy(table_hbm.at[idx_v.at[j]], rows_v.at[b],
                             sem_g.at[b])

        def wait_gather(j, b):
            pltpu.make_async_copy(table_hbm.at[idx_v.at[j]], rows_v.at[b],
                                  sem_g.at[b]).wait()

        def start_write(j, b):
            r, h = dst_slot(j)
            pltpu.async_copy(
                rows_v.at[b, :, pl.ds(0, _D)],
                out_hbm.at[pl.ds(r, _CHUNK), pl.ds(h * _D, _D)],
                sem_w.at[b])

        def wait_write(b):
            pltpu.make_async_copy(
                rows_v.at[b, :, pl.ds(0, _D)],
                out_hbm.at[pl.ds(0, _CHUNK), pl.ds(0, _D)],
                sem_w.at[b]).wait()

        for b in range(_NBUF):
            start_gather(b, b)

        def outer(g, carry):
            for b in range(_NBUF):
                j = g * _NBUF + b
                wait_gather(j, b)
                start_write(j, b)
            for b in range(_NBUF):
                jn = (g + 1) * _NBUF + b
                wait_write(b)
                start_gather(jn, b)
            return carry

        lax.fori_loop(0, n_outer - 1, outer, 0)

        gl = n_outer - 1
        for b in range(_NBUF):
            j = gl * _NBUF + b
            wait_gather(j, b)
            start_write(j, b)
        for b in range(_NBUF):
            wait_write(b)

    return body(idx2d, table)


def kernel(token_ids, weight):
    ids_flat = token_ids.T.astype(jnp.int32).reshape(-1)
    table = _pack_table(weight.T)
    g = _embed_lookup(ids_flat, table)
    out_t = _unpack_out(g.reshape(_S, _B // 2, 128))
    return jnp.transpose(out_t, (2, 0, 1))


# speedup vs baseline: 2.7699x; 1.1126x over previous
import functools

import jax
import jax.numpy as jnp
from jax import lax
from jax.experimental import pallas as pl
from jax.experimental.pallas import tpu as pltpu
from jax.experimental.pallas import tpu_sc as plsc

_D = 64
_CHUNK = 128
_NBUF = 5
_NW = 32
_V = 1000000
_VB = 16384
_TB = 16384
_S = 50
_B = 16384


def _pack_table(wt):
    def body(x_ref, y_ref):
        x = x_ref[...]
        xt = jnp.transpose(x)
        y_ref[...] = jnp.concatenate(
            [xt, jnp.zeros((_VB, 128 - _D), jnp.float32)], axis=1)

    return pl.pallas_call(
        body,
        grid=(pl.cdiv(_V, _VB),),
        in_specs=[pl.BlockSpec((_D, _VB), lambda j: (0, j))],
        out_specs=pl.BlockSpec((_VB, 128), lambda j: (j, 0)),
        out_shape=jax.ShapeDtypeStruct((_V, 128), jnp.float32),
    )(wt)


def _unpack_out(g3):
    def body(x_ref, y_ref):
        x = x_ref[0]
        y_ref[0] = jnp.concatenate(
            [jnp.transpose(x[:, :_D]), jnp.transpose(x[:, _D:])], axis=1)

    return pl.pallas_call(
        body,
        grid=(_S, _B // _TB),
        in_specs=[pl.BlockSpec((1, _TB // 2, 128), lambda s, j: (s, j, 0))],
        out_specs=pl.BlockSpec((1, _D, _TB), lambda s, j: (s, 0, j)),
        out_shape=jax.ShapeDtypeStruct((_S, _D, _B), jnp.float32),
    )(g3)


def _embed_lookup(flat_ids, table):
    B = flat_ids.shape[0]
    assert B % (_NW * _CHUNK * _NBUF) == 0
    n_chunks = B // (_NW * _CHUNK)
    n_outer = n_chunks // _NBUF
    idx2d = flat_ids.reshape(B // _CHUNK, _CHUNK)

    mesh = plsc.VectorSubcoreMesh(core_axis_name="c", subcore_axis_name="s")

    @functools.partial(
        pl.kernel,
        out_type=jax.ShapeDtypeStruct((B // 2, 128), jnp.float32),
        mesh=mesh,
        scratch_types=[
            pltpu.VMEM((n_chunks, _CHUNK), jnp.int32),
            pltpu.VMEM((_NBUF, _CHUNK, 128), jnp.float32),
            pltpu.SemaphoreType.DMA((_NBUF,)),
            pltpu.SemaphoreType.DMA((_NBUF,)),
        ],
        compiler_params=pltpu.CompilerParams(use_tc_tiling_on_sc=False),
    )
    def body(idx_hbm, table_hbm, out_hbm, idx_v, rows_v, sem_g, sem_w):
        wid = lax.axis_index("s") * 2 + lax.axis_index("c")
        row_base = wid * n_chunks
        pltpu.sync_copy(idx_hbm.at[pl.ds(row_base, n_chunks)], idx_v)

        def dst_slot(j):
            half = _TB // 2
            c = row_base + j
            s = c >> 7
            b0 = (c << 7) & (_B - 1)
            jblk = b0 // _TB
            k0 = b0 % half
            h = (b0 // half) & 1
            return s * (_B // 2) + jblk * half + k0, h

        def start_gather(j, b):
            pltpu.async_copy(table_hbm.at[idx_v.at[j]], rows_v.at[b],
                             sem_g.at[b])

        def wait_gather(j, b):
            pltpu.make_async_copy(table_hbm.at[idx_v.at[j]], rows_v.at[b],
                                  sem_g.at[b]).wait()

        def start_write(j, b):
            r, h = dst_slot(j)
            pltpu.async_copy(
                rows_v.at[b, :, pl.ds(0, _D)],
                out_hbm.at[pl.ds(r, _CHUNK), pl.ds(h * _D, _D)],
                sem_w.at[b])

        def wait_write(b):
            pltpu.make_async_copy(
                rows_v.at[b, :, pl.ds(0, _D)],
                out_hbm.at[pl.ds(0, _CHUNK), pl.ds(0, _D)],
                sem_w.at[b]).wait()

        for b in range(_NBUF):
            start_gather(b, b)

        def outer(g, carry):
            for b in range(_NBUF):
                j = g * _NBUF + b
                wait_gather(j, b)
                start_write(j, b)
            for b in range(_NBUF):
                jn = (g + 1) * _NBUF + b
                wait_write(b)
                start_gather(jn, b)
            return carry

        lax.fori_loop(0, n_outer - 1, outer, 0)

        gl = n_outer - 1
        for b in range(_NBUF):
            j = gl * _NBUF + b
            wait_gather(j, b)
            start_write(j, b)
        for b in range(_NBUF):
            wait_write(b)

    return body(idx2d, table)


def kernel(token_ids, weight):
    ids_flat = token_ids.T.astype(jnp.int32).reshape(-1)
    table = _pack_table(weight.T)
    g = _embed_lookup(ids_flat, table)
    out_t = _unpack_out(g.reshape(_S, _B // 2, 128))
    return jnp.transpose(out_t, (2, 0, 1))
